# Initial kernel scaffold; baseline (speedup 1.0000x reference)
import jax
import jax.numpy as jnp
from jax.experimental import pallas as pl


def kernel(y, table):
    raise NotImplementedError("write your pallas kernel here")


# speedup vs baseline: 1.8905x; 1.8905x over previous
import functools

import jax
import jax.numpy as jnp
from jax import lax
from jax.experimental import pallas as pl
from jax.experimental.pallas import tpu as pltpu
from jax.experimental.pallas import tpu_sc as plsc

B = 16384
L = 50
M = 64
N = B * L
NW = 32
CPW = N // NW
CHUNK = 512
NCH = CPW // CHUNK
TAIL = (N - L) - ((NW - 1) * CPW + (NCH - 1) * CHUNK)

_mesh = plsc.VectorSubcoreMesh(core_axis_name="c", subcore_axis_name="s")


@functools.partial(
    pl.kernel,
    mesh=_mesh,
    out_type=jax.ShapeDtypeStruct((N, M), jnp.float32),
    scratch_types=[
        pltpu.VMEM((CHUNK,), jnp.int32),
        pltpu.VMEM((CHUNK, M), jnp.float32),
        pltpu.VMEM((L, M), jnp.float32),
        pltpu.SemaphoreType.DMA,
    ],
    compiler_params=pltpu.CompilerParams(use_tc_tiling_on_sc=False),
)
def _emb_gather(idx_hbm, table_hbm, out_hbm, idx_v, rows_v, zbuf, sem):
    w = lax.axis_index("s") * 2 + lax.axis_index("c")
    base = w * CPW

    def body(c, carry):
        s = base + c * CHUNK
        pltpu.sync_copy(idx_hbm.at[pl.ds(s, CHUNK)], idx_v)
        pltpu.async_copy(table_hbm.at[idx_v], rows_v, sem).wait()
        pltpu.sync_copy(rows_v, out_hbm.at[pl.ds(s + L, CHUNK)])
        return carry

    nch = jnp.where(w == NW - 1, NCH - 1, NCH)
    lax.fori_loop(0, nch, body, 0)

    @pl.when(w == NW - 1)
    def _tail():
        s = base + (NCH - 1) * CHUNK
        pltpu.sync_copy(idx_hbm.at[pl.ds(s, CHUNK)], idx_v)
        pltpu.async_copy(table_hbm.at[idx_v], rows_v, sem).wait()
        pltpu.sync_copy(rows_v.at[pl.ds(0, TAIL)], out_hbm.at[pl.ds(s + L, TAIL)])

    @pl.when(w == 0)
    def _zero_head():
        def zrow(k, carry):
            zbuf[k // 4, pl.ds((k % 4) * 16, 16)] = jnp.zeros((16,), jnp.float32)
            return carry

        lax.fori_loop(0, L * 4, zrow, 0)
        pltpu.sync_copy(zbuf, out_hbm.at[pl.ds(0, L)])


def kernel(y, table):
    idx = y.reshape(-1).astype(jnp.int32)
    out = _emb_gather(idx, table)
    return out.reshape(B, L, M)


# speedup vs baseline: 1.9699x; 1.0420x over previous
import functools

import jax
import jax.numpy as jnp
from jax import lax
from jax.experimental import pallas as pl
from jax.experimental.pallas import tpu as pltpu
from jax.experimental.pallas import tpu_sc as plsc

B = 16384
L = 50
M = 64
N = B * L
NW = 32
CPW = N // NW
CHUNK = 512
NCH = CPW // CHUNK
TAIL = (N - L) - ((NW - 1) * CPW + (NCH - 1) * CHUNK)

_mesh = plsc.VectorSubcoreMesh(core_axis_name="c", subcore_axis_name="s")


@functools.partial(
    pl.kernel,
    mesh=_mesh,
    out_type=jax.ShapeDtypeStruct((N, M), jnp.float32),
    scratch_types=[
        pltpu.VMEM((CPW,), jnp.int32),
        pltpu.VMEM((2, CHUNK, M), jnp.float32),
        pltpu.VMEM((L, M), jnp.float32),
        pltpu.SemaphoreType.DMA((2,)),
        pltpu.SemaphoreType.DMA((2,)),
    ],
    compiler_params=pltpu.CompilerParams(use_tc_tiling_on_sc=False),
)
def _emb_gather(idx_hbm, table_hbm, out_hbm, idx_v, rows_v, zbuf, gsem, osem):
    w = lax.axis_index("s") * 2 + lax.axis_index("c")
    base = w * CPW
    nch = jnp.where(w == NW - 1, NCH - 1, NCH)

    pltpu.sync_copy(idx_hbm.at[pl.ds(base, CPW)], idx_v)

    def start_gather(c, b):
        pltpu.async_copy(
            table_hbm.at[idx_v.at[pl.ds(c * CHUNK, CHUNK)]],
            rows_v.at[b],
            gsem.at[b],
        )

    def wait_gather(b):
        pltpu.make_async_copy(
            table_hbm.at[idx_v.at[pl.ds(0, CHUNK)]], rows_v.at[b], gsem.at[b]
        ).wait()

    def wait_write(b):
        pltpu.make_async_copy(
            rows_v.at[b], out_hbm.at[pl.ds(base + L, CHUNK)], osem.at[b]
        ).wait()

    start_gather(0, 0)

    def body(c, carry):
        b = lax.rem(c, 2)
        nb = lax.rem(c + 1, 2)

        @pl.when(c + 1 < nch)
        def _():
            @pl.when(c >= 1)
            def _():
                wait_write(nb)

            start_gather(c + 1, nb)

        wait_gather(b)
        pltpu.async_copy(
            rows_v.at[b],
            out_hbm.at[pl.ds(base + c * CHUNK + L, CHUNK)],
            osem.at[b],
        )
        return carry

    lax.fori_loop(0, nch, body, 0)
    wait_write(0)
    wait_write(1)

    @pl.when(w == NW - 1)
    def _tail():
        s = (NCH - 1) * CHUNK
        pltpu.async_copy(
            table_hbm.at[idx_v.at[pl.ds(s, CHUNK)]], rows_v.at[0], gsem.at[0]
        ).wait()
        pltpu.sync_copy(
            rows_v.at[0, pl.ds(0, TAIL)], out_hbm.at[pl.ds(base + s + L, TAIL)]
        )

    @pl.when(w == 0)
    def _zero_head():
        def zrow(k, carry):
            zbuf[k // 4, pl.ds((k % 4) * 16, 16)] = jnp.zeros((16,), jnp.float32)
            return carry

        lax.fori_loop(0, L * 4, zrow, 0)
        pltpu.sync_copy(zbuf, out_hbm.at[pl.ds(0, L)])


def kernel(y, table):
    idx = y.reshape(-1).astype(jnp.int32)
    out = _emb_gather(idx, table)
    return out.reshape(B, L, M)


# speedup vs baseline: 2.0612x; 1.0464x over previous
import functools

import jax
import jax.numpy as jnp
from jax import lax
from jax.experimental import pallas as pl
from jax.experimental.pallas import tpu as pltpu
from jax.experimental.pallas import tpu_sc as plsc

B = 16384
L = 50
M = 64
N = B * L
NW = 32
CPW = N // NW
CHUNK = 512
NCH = CPW // CHUNK

_mesh = plsc.VectorSubcoreMesh(core_axis_name="c", subcore_axis_name="s")


@functools.partial(
    pl.kernel,
    mesh=_mesh,
    out_type=jax.ShapeDtypeStruct((N, M), jnp.float32),
    scratch_types=[
        pltpu.VMEM((CPW,), jnp.int32),
        pltpu.VMEM((2, CHUNK, M), jnp.float32),
        pltpu.SemaphoreType.DMA((2,)),
        pltpu.SemaphoreType.DMA((2,)),
    ],
    compiler_params=pltpu.CompilerParams(use_tc_tiling_on_sc=False),
)
def _emb_gather(z_hbm, table_hbm, out_hbm, idx_v, rows_v, gsem, osem):
    w = lax.axis_index("s") * 2 + lax.axis_index("c")
    base = w * CPW

    pltpu.sync_copy(z_hbm.at[pl.ds(base, CPW)], idx_v)

    def start_gather(c, b):
        pltpu.async_copy(
            table_hbm.at[idx_v.at[pl.ds(c * CHUNK, CHUNK)]],
            rows_v.at[b],
            gsem.at[b],
        )

    def wait_gather(b):
        pltpu.make_async_copy(
            table_hbm.at[idx_v.at[pl.ds(0, CHUNK)]], rows_v.at[b], gsem.at[b]
        ).wait()

    def wait_write(b):
        pltpu.make_async_copy(
            rows_v.at[b], out_hbm.at[pl.ds(base, CHUNK)], osem.at[b]
        ).wait()

    start_gather(0, 0)

    def body(c, carry):
        b = lax.rem(c, 2)
        nb = lax.rem(c + 1, 2)
        s = base + c * CHUNK

        @pl.when(c + 1 < NCH)
        def _():
            @pl.when(c >= 1)
            def _():
                wait_write(nb)

            start_gather(c + 1, nb)

        wait_gather(b)

        @pl.when(lax.rem(s, B) == 0)
        def _():
            for j in range(M // 16):
                rows_v[b, 0, pl.ds(j * 16, 16)] = jnp.zeros((16,), jnp.float32)

        pltpu.async_copy(
            rows_v.at[b],
            out_hbm.at[pl.ds(s, CHUNK)],
            osem.at[b],
        )
        return carry

    lax.fori_loop(0, NCH, body, 0)
    wait_write(0)
    wait_write(1)


def kernel(y, table):
    y32 = y.astype(jnp.int32)
    zt = jnp.concatenate([jnp.zeros((1, L), jnp.int32), y32[: B - 1]], axis=0)
    zt = zt.T.reshape(-1)
    out = _emb_gather(zt, table)
    return out.reshape(L, B, M).transpose(1, 0, 2)
